# Initial kernel scaffold; baseline (speedup 1.0000x reference)
import jax
import jax.numpy as jnp
from jax.experimental import pallas as pl


def kernel(s, a, W1, b1, Wg, bg, W2, b2, Wq1a, bq1a, Wq1b, bq1b, Wq2a, bq2a, Wq2b, bq2b):
    raise NotImplementedError("write your pallas kernel here")


# speedup vs baseline: 8.7887x; 8.7887x over previous
import functools

import jax
import jax.numpy as jnp
from jax.experimental import pallas as pl
from jax.experimental.pallas import tpu as pltpu

_BLOCK = 2000
_OBS = 32
_ACT = 16


def _body(s_ref, a_ref, W1_ref, b1_ref, Wg_ref, bg_ref, W2_ref, b2_ref,
          Wq1a_ref, bq1a_ref, Wq1b_ref, bq1b_ref,
          Wq2a_ref, bq2a_ref, Wq2b_ref, bq2b_ref,
          q1_ref, q2_ref):
    acc = jnp.dot(s_ref[0], W1_ref[0:_OBS, :])
    acc += jnp.dot(s_ref[1], W1_ref[_OBS:2 * _OBS, :])
    acc += jnp.dot(s_ref[2], W1_ref[2 * _OBS:3 * _OBS, :])
    off = 3 * _OBS
    acc += jnp.dot(a_ref[0], W1_ref[off:off + _ACT, :])
    acc += jnp.dot(a_ref[1], W1_ref[off + _ACT:off + 2 * _ACT, :])
    acc += jnp.dot(a_ref[2], W1_ref[off + 2 * _ACT:off + 3 * _ACT, :])
    fc1 = jnp.maximum(acc + b1_ref[:], 0.0)

    xw = jnp.dot(fc1, Wg_ref[:])
    clique = (xw[0:1, :] + xw[1:2, :] + xw[2:3, :]) * (1.0 / 3.0)
    row = jax.lax.broadcasted_iota(jnp.int32, (_BLOCK, 1), 0)
    in_clique = jnp.logical_and(pl.program_id(0) == 0, row < 3)
    xw = jnp.where(in_clique, clique, xw)
    g = jnp.maximum(xw + bg_ref[:], 0.0) + fc1

    fc2 = jnp.maximum(jnp.dot(g, W2_ref[:]) + b2_ref[:], 0.0)

    h1 = jnp.maximum(jnp.dot(fc2, Wq1a_ref[:]) + bq1a_ref[:], 0.0)
    q1_ref[:] = jnp.dot(h1, Wq1b_ref[:]) + bq1b_ref[:]
    h2 = jnp.maximum(jnp.dot(fc2, Wq2a_ref[:]) + bq2a_ref[:], 0.0)
    q2_ref[:] = jnp.dot(h2, Wq2b_ref[:]) + bq2b_ref[:]


@functools.partial(jax.jit, static_argnames=("interpret",))
def kernel(s, a, W1, b1, Wg, bg, W2, b2, Wq1a, bq1a, Wq1b, bq1b,
           Wq2a, bq2a, Wq2b, bq2b, interpret=False):
    na, n, obs = s.shape
    h = W1.shape[1]
    grid = (n // _BLOCK,)

    def rows(i):
        return (0, i, 0)

    def full(i):
        return (0, 0)

    in_specs = [
        pl.BlockSpec((na, _BLOCK, obs), rows),
        pl.BlockSpec((na, _BLOCK, a.shape[2]), rows),
        pl.BlockSpec(W1.shape, full),
        pl.BlockSpec((1, h), full),
        pl.BlockSpec(Wg.shape, full),
        pl.BlockSpec((1, h), full),
        pl.BlockSpec(W2.shape, full),
        pl.BlockSpec((1, h), full),
        pl.BlockSpec(Wq1a.shape, full),
        pl.BlockSpec((1, h), full),
        pl.BlockSpec(Wq1b.shape, full),
        pl.BlockSpec((1, 1), full),
        pl.BlockSpec(Wq2a.shape, full),
        pl.BlockSpec((1, h), full),
        pl.BlockSpec(Wq2b.shape, full),
        pl.BlockSpec((1, 1), full),
    ]
    out_specs = [
        pl.BlockSpec((_BLOCK, 1), lambda i: (i, 0)),
        pl.BlockSpec((_BLOCK, 1), lambda i: (i, 0)),
    ]
    q1, q2 = pl.pallas_call(
        _body,
        grid=grid,
        in_specs=in_specs,
        out_specs=out_specs,
        out_shape=[jax.ShapeDtypeStruct((n, 1), jnp.float32)] * 2,
        compiler_params=pltpu.CompilerParams(
            dimension_semantics=("parallel",)),
        interpret=interpret,
    )(s, a, W1, b1.reshape(1, h), Wg, bg.reshape(1, h), W2, b2.reshape(1, h),
      Wq1a, bq1a.reshape(1, h), Wq1b, bq1b.reshape(1, 1),
      Wq2a, bq2a.reshape(1, h), Wq2b, bq2b.reshape(1, 1))
    return (q1, q2)
